# baseline (speedup 1.0000x reference)
import jax
import jax.numpy as jnp
from jax.experimental import pallas as pl

_B, _NPOS, _NNEG, _D, _NLEM = 8, 128, 64, 128, 16384
_MOM = 0.5
_T = _NLEM // _B


def _renorm(m):
    s = jnp.sum(m * m, axis=1, keepdims=True)
    return m / jnp.maximum(jnp.sqrt(s), 1e-12)


def _body(x_ref, y_ref, vis_ref, mem_ref,
          sim_ref, noise_ref, newmem_ref, acc_ref, y_out_ref):
    i = pl.program_id(0)
    mem = mem_ref[...]
    xb = x_ref[i, 0:_NPOS, :]
    sim_ref[...] = jax.lax.dot_general(
        xb, mem, (((1,), (1,)), ((), ())), preferred_element_type=jnp.float32)

    r0 = pl.multiple_of(i * _T, _T)
    mtile = mem_ref[pl.ds(r0, _T), :]

    @pl.when(i == 0)
    def _():
        vis = vis_ref[...]
        x3 = x_ref[:, 0:_NPOS, :]
        xneg = x_ref[:, _NPOS:, :].reshape(_B * _NNEG, _D)
        mem_head = mem[0:_NPOS, :]
        noise_ref[...] = jax.lax.dot_general(
            xneg, mem_head, (((1,), (1,)), ((), ())),
            preferred_element_type=jnp.float32)
        get = jnp.mean(x3 * vis[..., None], axis=0)
        head = mem_head * _MOM + get * (1.0 - _MOM)
        newmem_ref[0:_NPOS, :] = _renorm(head)
        newmem_ref[_NPOS:_NPOS + _B * _NNEG, :] = _renorm(xneg)
        newmem_ref[_NPOS + _B * _NNEG:, :] = _renorm(
            mtile[_NPOS + _B * _NNEG:, :])
        acc_ref[...] = jnp.sum((vis > 0).astype(jnp.int32), axis=0,
                               keepdims=True)
        y_out_ref[...] = y_ref[...]

    @pl.when(i != 0)
    def _():
        newmem_ref[...] = _renorm(mtile)


def kernel(x, y, visible, memory):
    sim, noise, new_memory, acc, y_idx = pl.pallas_call(
        _body,
        grid=(_B,),
        in_specs=[
            pl.BlockSpec((_B, _NPOS + _NNEG, _D), lambda i: (0, 0, 0)),
            pl.BlockSpec((_B, _NPOS), lambda i: (0, 0)),
            pl.BlockSpec((_B, _NPOS), lambda i: (0, 0)),
            pl.BlockSpec((_NLEM, _D), lambda i: (0, 0)),
        ],
        out_specs=[
            pl.BlockSpec((_NPOS, _NLEM), lambda i: (i, 0)),
            pl.BlockSpec((_B * _NNEG, _NPOS), lambda i: (0, 0)),
            pl.BlockSpec((_T, _D), lambda i: (i, 0)),
            pl.BlockSpec((1, _NPOS), lambda i: (0, 0)),
            pl.BlockSpec((_B, _NPOS), lambda i: (0, 0)),
        ],
        out_shape=[
            jax.ShapeDtypeStruct((_B * _NPOS, _NLEM), jnp.float32),
            jax.ShapeDtypeStruct((_B * _NNEG, _NPOS), jnp.float32),
            jax.ShapeDtypeStruct((_NLEM, _D), jnp.float32),
            jax.ShapeDtypeStruct((1, _NPOS), jnp.int32),
            jax.ShapeDtypeStruct((_B, _NPOS), jnp.int32),
        ],
    )(x, y.astype(jnp.int32), visible, memory)

    similarity = sim.reshape(_B, _NPOS, _NLEM)
    noise_similarity = noise.reshape(_B, _NNEG, _NPOS)
    accumulate_delta = acc.reshape(_NPOS)
    return (similarity, y_idx, noise_similarity, new_memory, accumulate_delta)


# speedup vs baseline: 1.0440x; 1.0440x over previous
import jax
import jax.numpy as jnp
from jax.experimental import pallas as pl

_B, _NPOS, _NNEG, _D, _NLEM = 8, 128, 64, 128, 16384
_MOM = 0.5
_T = 2048
_GRID = _NLEM // _T


def _renorm(m):
    s = jnp.sum(m * m, axis=1, keepdims=True)
    return m * jnp.minimum(jax.lax.rsqrt(s), 1e12)


def _body(x_ref, y_ref, vis_ref, mem_ref,
          sim_ref, noise_ref, newmem_ref, acc_ref, y_out_ref):
    i = pl.program_id(0)
    x3 = x_ref[:, 0:_NPOS, :]
    xf = x3.reshape(_B * _NPOS, _D)
    mem = mem_ref[...]
    sim_ref[...] = jax.lax.dot_general(
        xf, mem, (((1,), (1,)), ((), ())), preferred_element_type=jnp.float32)

    @pl.when(i == 0)
    def _():
        vis = vis_ref[...]
        xneg = x_ref[:, _NPOS:, :].reshape(_B * _NNEG, _D)
        mem_head = mem[0:_NPOS, :]
        noise_ref[...] = jax.lax.dot_general(
            xneg, mem_head, (((1,), (1,)), ((), ())),
            preferred_element_type=jnp.float32)
        get = jnp.mean(x3 * vis[..., None], axis=0)
        head = mem_head * _MOM + get * (1.0 - _MOM)
        newmem_ref[0:_NPOS, :] = _renorm(head)
        newmem_ref[_NPOS:_NPOS + _B * _NNEG, :] = _renorm(xneg)
        newmem_ref[_NPOS + _B * _NNEG:, :] = _renorm(mem[_NPOS + _B * _NNEG:, :])
        acc_ref[...] = jnp.sum((vis > 0).astype(jnp.int32), axis=0,
                               keepdims=True)
        y_out_ref[...] = y_ref[...]

    @pl.when(i != 0)
    def _():
        newmem_ref[...] = _renorm(mem)


def kernel(x, y, visible, memory):
    sim, noise, new_memory, acc, y_idx = pl.pallas_call(
        _body,
        grid=(_GRID,),
        in_specs=[
            pl.BlockSpec((_B, _NPOS + _NNEG, _D), lambda i: (0, 0, 0)),
            pl.BlockSpec((_B, _NPOS), lambda i: (0, 0)),
            pl.BlockSpec((_B, _NPOS), lambda i: (0, 0)),
            pl.BlockSpec((_T, _D), lambda i: (i, 0)),
        ],
        out_specs=[
            pl.BlockSpec((_B * _NPOS, _T), lambda i: (0, i)),
            pl.BlockSpec((_B * _NNEG, _NPOS), lambda i: (0, 0)),
            pl.BlockSpec((_T, _D), lambda i: (i, 0)),
            pl.BlockSpec((1, _NPOS), lambda i: (0, 0)),
            pl.BlockSpec((_B, _NPOS), lambda i: (0, 0)),
        ],
        out_shape=[
            jax.ShapeDtypeStruct((_B * _NPOS, _NLEM), jnp.float32),
            jax.ShapeDtypeStruct((_B * _NNEG, _NPOS), jnp.float32),
            jax.ShapeDtypeStruct((_NLEM, _D), jnp.float32),
            jax.ShapeDtypeStruct((1, _NPOS), jnp.int32),
            jax.ShapeDtypeStruct((_B, _NPOS), jnp.int32),
        ],
    )(x, y.astype(jnp.int32), visible, memory)

    similarity = sim.reshape(_B, _NPOS, _NLEM)
    noise_similarity = noise.reshape(_B, _NNEG, _NPOS)
    accumulate_delta = acc.reshape(_NPOS)
    return (similarity, y_idx, noise_similarity, new_memory, accumulate_delta)
